# baseline (speedup 1.0000x reference)
import dataclasses
import functools

import jax
import jax.numpy as jnp
from jax import lax
from jax.experimental import pallas as pl
from jax.experimental.pallas import tpu as pltpu
from jax.experimental.pallas import tpu_sc as plsc

N = 4096
NC = 2
NS = 16
L = 16
NW = NC * NS
QPW = N // NW
NB = 192
NCH = N // L
BPW = QPW // L
BIG = 1e18


def _knn_sc(pts_t):
    mesh = plsc.VectorSubcoreMesh(core_axis_name="c", subcore_axis_name="s")
    cp = pltpu.CompilerParams()
    if "needs_layout_passes" in pltpu.CompilerParams.__dataclass_fields__:
        cp = dataclasses.replace(cp, needs_layout_passes=False)

    @functools.partial(
        pl.kernel,
        mesh=mesh,
        compiler_params=cp,
        out_type=jax.ShapeDtypeStruct((N, 128), jnp.float32),
        scratch_types=[
            pltpu.VMEM((3, N), jnp.float32),
            pltpu.VMEM((N,), jnp.float32),
            pltpu.VMEM((N,), jnp.float32),
            pltpu.VMEM((N,), jnp.float32),
            pltpu.VMEM((N,), jnp.float32),
            pltpu.VMEM((N,), jnp.int32),
            pltpu.VMEM((N,), jnp.int32),
            pltpu.VMEM((NB * L,), jnp.int32),
            pltpu.VMEM((NB + L,), jnp.int32),
            pltpu.VMEM((QPW, 128), jnp.float32),
            pltpu.VMEM((QPW,), jnp.int32),
        ],
    )
    def knn(pts_hbm, out_hbm, pts_v, sx_v, sy_v, sz_v, spp_v, soi_v, bidx_v,
            cnt_v, s_v, rows_v, oi_v):
        wid = lax.axis_index("s") * NC + lax.axis_index("c")
        pltpu.sync_copy(pts_hbm, pts_v)
        lane = lax.iota(jnp.int32, L)
        inf = jnp.full((L,), jnp.inf, jnp.float32)
        ones_i = jnp.full((L,), 1, jnp.int32)

        def minmax_body(c, carry):
            mn, mx = carry
            v = pts_v[0, pl.ds(c * L, L)]
            return jnp.minimum(mn, v), jnp.maximum(mx, v)

        mn, mx = lax.fori_loop(0, NCH, minmax_body, (inf, -inf))
        xmin = jnp.full((L,), jnp.min(mn), jnp.float32)
        xmax = jnp.full((L,), jnp.max(mx), jnp.float32)
        denom = jnp.maximum(xmax - xmin, 1e-30)
        invw = jnp.float32(NB) / denom
        w = denom / jnp.float32(NB)
        margin = w * jnp.float32(1e-3)

        def zero_body(c, _):
            cnt_v[pl.ds(c * L, L)] = jnp.zeros((L,), jnp.int32)
            return 0

        lax.fori_loop(0, NB, zero_body, 0)

        def hist_body(c, _):
            xv = pts_v[0, pl.ds(c * L, L)]
            bi = ((xv - xmin) * invw).astype(jnp.int32)
            bi = jnp.minimum(jnp.maximum(bi, 0), NB - 1)
            bidx_v[pl.ds(c * L, L)] = bi
            plsc.addupdate_scatter(cnt_v, [bi * L + lane], ones_i)
            return 0

        lax.fori_loop(0, NCH, hist_body, 0)

        def pfx_body(c, run):
            v = cnt_v[pl.ds(c * L, L)]
            cs = plsc.cumsum(v)
            cnt_v[pl.ds(c * L, L)] = cs - v + run
            return run + cs[L - 1]

        lax.fori_loop(0, NB, pfx_body, jnp.int32(0))

        def sgather_body(g, _):
            idx = (g * L + lane) * L
            s_v[pl.ds(g * L, L)] = plsc.load_gather(cnt_v, [idx])
            return 0

        lax.fori_loop(0, NB // L, sgather_body, 0)
        s_v[pl.ds(NB, L)] = jnp.full((L,), N, jnp.int32)

        def place_body(c, _):
            bi = bidx_v[pl.ds(c * L, L)]
            cidx = bi * L + lane
            pos = plsc.load_gather(cnt_v, [cidx])
            px = pts_v[0, pl.ds(c * L, L)]
            py = pts_v[1, pl.ds(c * L, L)]
            pz = pts_v[2, pl.ds(c * L, L)]
            plsc.store_scatter(sx_v, [pos], px)
            plsc.store_scatter(sy_v, [pos], py)
            plsc.store_scatter(sz_v, [pos], pz)
            plsc.store_scatter(spp_v, [pos], px * px + py * py + pz * pz)
            plsc.store_scatter(soi_v, [pos], c * L + lane)
            plsc.store_scatter(cnt_v, [cidx], pos + 1)
            return 0

        lax.fori_loop(0, NCH, place_body, 0)

        def read_s(b):
            return plsc.load_gather(s_v, [jnp.full((L,), b, jnp.int32)])[0]

        def block_body(i, _):
            base = (i * NW + wid) * L
            qx = sx_v[pl.ds(base, L)]
            qy = sy_v[pl.ds(base, L)]
            qz = sz_v[pl.ds(base, L)]
            qlo = jnp.full((L,), jnp.min(qx), jnp.float32)
            qhi = jnp.full((L,), jnp.max(qx), jnp.float32)
            qx2 = qx * (-2.0)
            qy2 = qy * (-2.0)
            qz2 = qz * (-2.0)
            qq = qx * qx + qy * qy + qz * qz

            def insert3(t, d2):
                t1, t2, t3 = t
                b1 = jnp.maximum(t1, d2)
                t1 = jnp.minimum(t1, d2)
                b2 = jnp.maximum(t2, b1)
                t2 = jnp.minimum(t2, b1)
                t3 = jnp.minimum(t3, b2)
                return t1, t2, t3

            def scan(c0, c1, t):
                def chunk_body(c, t):
                    px = sx_v[pl.ds(c * L, L)]
                    py = sy_v[pl.ds(c * L, L)]
                    pz = sz_v[pl.ds(c * L, L)]
                    pp = spp_v[pl.ds(c * L, L)]
                    for k in range(L):
                        d2 = qx2 * px[k] + qy2 * py[k] + qz2 * pz[k] + pp[k]
                        t = insert3(t, d2)
                    return t

                return lax.fori_loop(c0, c1, chunk_body, t)

            def self_chunk(t):
                for k in range(L):
                    d2 = qx2 * qx[k] + qy2 * qy[k] + qz2 * qz[k] + qq[k]
                    d2 = jnp.where(lane == k, BIG, d2)
                    t = insert3(t, d2)
                return t

            def lane_bound(lo, hi):
                lov = jnp.broadcast_to(lo, (L,))
                hiv = jnp.broadcast_to(hi, (L,))
                edge_l = xmin + lov.astype(jnp.float32) * w
                edge_r = xmin + hiv.astype(jnp.float32) * w
                gap_l = jnp.where(lov == 0, BIG, qx - edge_l - margin)
                gap_r = jnp.where(hiv == NB, BIG, edge_r - qx - margin)
                g = jnp.maximum(jnp.minimum(gap_l, gap_r), 0.0)
                return g * g - qq

            lo0 = jnp.maximum(
                jnp.minimum(((qlo - xmin) * invw).astype(jnp.int32)[0],
                            NB - 1) - 1, 0)
            hi0 = jnp.minimum(
                jnp.minimum(((qhi - xmin) * invw).astype(jnp.int32)[0],
                            NB - 1) + 2, NB)
            lc0 = lax.shift_right_logical(read_s(lo0), 4)
            rc0 = jnp.minimum(
                lax.shift_right_logical(read_s(hi0) + (L - 1), 4), NCH)
            blk = i * NW + wid
            t = scan(lc0, blk, (inf, inf, inf))
            t = self_chunk(t)
            t = scan(blk + 1, rc0, t)

            def w_cond(carry):
                t1, t2, t3, lo, hi, lc, rc = carry
                return jnp.any(t3 > lane_bound(lo, hi))

            def w_body(carry):
                t1, t2, t3, lo, hi, lc, rc = carry
                nlo = jnp.maximum(lo - 1, 0)
                nhi = jnp.minimum(hi + 1, NB)
                nlc = lax.shift_right_logical(read_s(nlo), 4)
                nrc = jnp.minimum(
                    lax.shift_right_logical(read_s(nhi) + (L - 1), 4), NCH)
                t = scan(nlc, lc, (t1, t2, t3))
                t = scan(rc, nrc, t)
                return t + (nlo, nhi, nlc, nrc)

            t1, t2, t3, _, _, _, _ = lax.while_loop(
                w_cond, w_body, t + (lo0, hi0, lc0, rc0))

            row = i * L + lane
            plsc.store_scatter(rows_v, [row, jnp.zeros((L,), jnp.int32)],
                               t1 + qq)
            plsc.store_scatter(rows_v, [row, jnp.full((L,), 1, jnp.int32)],
                               t2 + qq)
            plsc.store_scatter(rows_v, [row, jnp.full((L,), 2, jnp.int32)],
                               t3 + qq)
            oi_v[pl.ds(i * L, L)] = soi_v[pl.ds(base, L)]
            return 0

        lax.fori_loop(0, BPW, block_body, 0)
        pltpu.sync_copy(rows_v, out_hbm.at[oi_v])

    return knn(pts_t)


def _cov_body(d_ref, s_ref, q_ref, o_ref):
    d1 = jnp.sqrt(d_ref[0:1, :])
    d2 = jnp.sqrt(d_ref[1:2, :])
    d3 = jnp.sqrt(d_ref[2:3, :])
    asc = jnp.maximum((d1 + d2 + d3) / 3.0, 1e-5)
    r = q_ref[0:1, :]
    x = q_ref[1:2, :]
    y = q_ref[2:3, :]
    z = q_ref[3:4, :]
    inv = 1.0 / jnp.sqrt(r * r + x * x + y * y + z * z)
    r = r * inv
    x = x * inv
    y = y * inv
    z = z * inv
    rot = (
        (1.0 - 2.0 * (y * y + z * z), 2.0 * (x * y - r * z), 2.0 * (x * z + r * y)),
        (2.0 * (x * y + r * z), 1.0 - 2.0 * (x * x + z * z), 2.0 * (y * z - r * x)),
        (2.0 * (x * z - r * y), 2.0 * (y * z + r * x), 1.0 - 2.0 * (x * x + y * y)),
    )
    p = tuple((s_ref[k : k + 1, :] * asc) ** 2 for k in range(3))
    for a in range(3):
        for b in range(a, 3):
            cov = p[0] * rot[a][0] * rot[b][0] + p[1] * rot[a][1] * rot[b][1] + p[2] * rot[a][2] * rot[b][2]
            o_ref[3 * a + b : 3 * a + b + 1, :] = cov
            if a != b:
                o_ref[3 * b + a : 3 * b + a + 1, :] = cov


def _cov_tc(d2top, scales_t, quats_t):
    return pl.pallas_call(
        _cov_body,
        out_shape=jax.ShapeDtypeStruct((9, N), jnp.float32),
    )(d2top, scales_t, quats_t)


def kernel(points, colors, scales, quaternions):
    del colors
    pts_t = points.T
    d2rows = _knn_sc(pts_t)
    d2top = d2rows[:, :3].T
    covt = _cov_tc(d2top, scales.T, quaternions.T)
    return covt.T.reshape(N, 3, 3)


# speedup vs baseline: 1.0573x; 1.0573x over previous
import dataclasses
import functools

import jax
import jax.numpy as jnp
from jax import lax
from jax.experimental import pallas as pl
from jax.experimental.pallas import tpu as pltpu
from jax.experimental.pallas import tpu_sc as plsc

N = 4096
NC = 2
NS = 16
L = 16
NW = NC * NS
QPW = N // NW
NB = 96
NCH = N // L
BPW = QPW // L
BIG = 1e18


def _knn_sc(pts_t):
    mesh = plsc.VectorSubcoreMesh(core_axis_name="c", subcore_axis_name="s")
    cp = pltpu.CompilerParams()
    if "needs_layout_passes" in pltpu.CompilerParams.__dataclass_fields__:
        cp = dataclasses.replace(cp, needs_layout_passes=False)

    @functools.partial(
        pl.kernel,
        mesh=mesh,
        compiler_params=cp,
        out_type=jax.ShapeDtypeStruct((N, 128), jnp.float32),
        scratch_types=[
            pltpu.VMEM((3, N), jnp.float32),
            pltpu.VMEM((N,), jnp.float32),
            pltpu.VMEM((N,), jnp.float32),
            pltpu.VMEM((N,), jnp.float32),
            pltpu.VMEM((N,), jnp.float32),
            pltpu.VMEM((N,), jnp.int32),
            pltpu.VMEM((N,), jnp.int32),
            pltpu.VMEM((NB * L,), jnp.int32),
            pltpu.VMEM((NB + L,), jnp.int32),
            pltpu.VMEM((QPW, 128), jnp.float32),
            pltpu.VMEM((QPW,), jnp.int32),
        ],
    )
    def knn(pts_hbm, out_hbm, pts_v, sx_v, sy_v, sz_v, spp_v, soi_v, bidx_v,
            cnt_v, s_v, rows_v, oi_v):
        wid = lax.axis_index("s") * NC + lax.axis_index("c")
        pltpu.sync_copy(pts_hbm, pts_v)
        lane = lax.iota(jnp.int32, L)
        inf = jnp.full((L,), jnp.inf, jnp.float32)
        ones_i = jnp.full((L,), 1, jnp.int32)

        def minmax_body(c, carry):
            mn, mx = carry
            v = pts_v[0, pl.ds(c * L, L)]
            return jnp.minimum(mn, v), jnp.maximum(mx, v)

        mn, mx = lax.fori_loop(0, NCH, minmax_body, (inf, -inf))
        xmin = jnp.full((L,), jnp.min(mn), jnp.float32)
        xmax = jnp.full((L,), jnp.max(mx), jnp.float32)
        denom = jnp.maximum(xmax - xmin, 1e-30)
        invw = jnp.float32(NB) / denom
        w = denom / jnp.float32(NB)
        margin = w * jnp.float32(1e-3)

        def zero_body(c, _):
            cnt_v[pl.ds(c * L, L)] = jnp.zeros((L,), jnp.int32)
            return 0

        lax.fori_loop(0, NB, zero_body, 0)

        def hist_body(c, _):
            xv = pts_v[0, pl.ds(c * L, L)]
            bi = ((xv - xmin) * invw).astype(jnp.int32)
            bi = jnp.minimum(jnp.maximum(bi, 0), NB - 1)
            bidx_v[pl.ds(c * L, L)] = bi
            plsc.addupdate_scatter(cnt_v, [bi * L + lane], ones_i)
            return 0

        lax.fori_loop(0, NCH, hist_body, 0)

        def pfx_body(c, run):
            v = cnt_v[pl.ds(c * L, L)]
            cs = plsc.cumsum(v)
            cnt_v[pl.ds(c * L, L)] = cs - v + run
            return run + cs[L - 1]

        lax.fori_loop(0, NB, pfx_body, jnp.int32(0))

        def sgather_body(g, _):
            idx = (g * L + lane) * L
            s_v[pl.ds(g * L, L)] = plsc.load_gather(cnt_v, [idx])
            return 0

        lax.fori_loop(0, NB // L, sgather_body, 0)
        s_v[pl.ds(NB, L)] = jnp.full((L,), N, jnp.int32)

        def place_body(c, _):
            bi = bidx_v[pl.ds(c * L, L)]
            cidx = bi * L + lane
            pos = plsc.load_gather(cnt_v, [cidx])
            px = pts_v[0, pl.ds(c * L, L)]
            py = pts_v[1, pl.ds(c * L, L)]
            pz = pts_v[2, pl.ds(c * L, L)]
            plsc.store_scatter(sx_v, [pos], px)
            plsc.store_scatter(sy_v, [pos], py)
            plsc.store_scatter(sz_v, [pos], pz)
            plsc.store_scatter(spp_v, [pos], px * px + py * py + pz * pz)
            plsc.store_scatter(soi_v, [pos], c * L + lane)
            plsc.store_scatter(cnt_v, [cidx], pos + 1)
            return 0

        lax.fori_loop(0, NCH, place_body, 0)

        def read_s(b):
            return plsc.load_gather(s_v, [jnp.full((L,), b, jnp.int32)])[0]

        def block_body(i, _):
            base = (i * NW + wid) * L
            qx = sx_v[pl.ds(base, L)]
            qy = sy_v[pl.ds(base, L)]
            qz = sz_v[pl.ds(base, L)]
            qlo = jnp.full((L,), jnp.min(qx), jnp.float32)
            qhi = jnp.full((L,), jnp.max(qx), jnp.float32)
            qx2 = qx * (-2.0)
            qy2 = qy * (-2.0)
            qz2 = qz * (-2.0)
            qq = qx * qx + qy * qy + qz * qz

            def insert3(t, d2):
                t1, t2, t3 = t
                b1 = jnp.maximum(t1, d2)
                t1 = jnp.minimum(t1, d2)
                b2 = jnp.maximum(t2, b1)
                t2 = jnp.minimum(t2, b1)
                t3 = jnp.minimum(t3, b2)
                return t1, t2, t3

            def scan(c0, c1, t):
                def chunk_body(c, t):
                    px = sx_v[pl.ds(c * L, L)]
                    py = sy_v[pl.ds(c * L, L)]
                    pz = sz_v[pl.ds(c * L, L)]
                    pp = spp_v[pl.ds(c * L, L)]
                    for k in range(L):
                        d2 = qx2 * px[k] + qy2 * py[k] + qz2 * pz[k] + pp[k]
                        t = insert3(t, d2)
                    return t

                return lax.fori_loop(c0, c1, chunk_body, t)

            def self_chunk(t):
                for k in range(L):
                    d2 = qx2 * qx[k] + qy2 * qy[k] + qz2 * qz[k] + qq[k]
                    d2 = jnp.where(lane == k, BIG, d2)
                    t = insert3(t, d2)
                return t

            def lane_bound(lo, hi):
                lov = jnp.broadcast_to(lo, (L,))
                hiv = jnp.broadcast_to(hi, (L,))
                edge_l = xmin + lov.astype(jnp.float32) * w
                edge_r = xmin + hiv.astype(jnp.float32) * w
                gap_l = jnp.where(lov == 0, BIG, qx - edge_l - margin)
                gap_r = jnp.where(hiv == NB, BIG, edge_r - qx - margin)
                g = jnp.maximum(jnp.minimum(gap_l, gap_r), 0.0)
                return g * g - qq

            lo0 = jnp.maximum(
                jnp.minimum(((qlo - xmin) * invw).astype(jnp.int32)[0],
                            NB - 1) - 1, 0)
            hi0 = jnp.minimum(
                jnp.minimum(((qhi - xmin) * invw).astype(jnp.int32)[0],
                            NB - 1) + 2, NB)
            lc0 = lax.shift_right_logical(read_s(lo0), 4)
            rc0 = jnp.minimum(
                lax.shift_right_logical(read_s(hi0) + (L - 1), 4), NCH)
            blk = i * NW + wid
            t = scan(lc0, blk, (inf, inf, inf))
            t = self_chunk(t)
            t = scan(blk + 1, rc0, t)

            def w_cond(carry):
                t1, t2, t3, lo, hi, lc, rc = carry
                return jnp.any(t3 > lane_bound(lo, hi))

            def w_body(carry):
                t1, t2, t3, lo, hi, lc, rc = carry
                nlo = jnp.maximum(lo - 1, 0)
                nhi = jnp.minimum(hi + 1, NB)
                nlc = lax.shift_right_logical(read_s(nlo), 4)
                nrc = jnp.minimum(
                    lax.shift_right_logical(read_s(nhi) + (L - 1), 4), NCH)
                t = scan(nlc, lc, (t1, t2, t3))
                t = scan(rc, nrc, t)
                return t + (nlo, nhi, nlc, nrc)

            t1, t2, t3, _, _, _, _ = lax.while_loop(
                w_cond, w_body, t + (lo0, hi0, lc0, rc0))

            row = i * L + lane
            plsc.store_scatter(rows_v, [row, jnp.zeros((L,), jnp.int32)],
                               t1 + qq)
            plsc.store_scatter(rows_v, [row, jnp.full((L,), 1, jnp.int32)],
                               t2 + qq)
            plsc.store_scatter(rows_v, [row, jnp.full((L,), 2, jnp.int32)],
                               t3 + qq)
            oi_v[pl.ds(i * L, L)] = soi_v[pl.ds(base, L)]
            return 0

        lax.fori_loop(0, BPW, block_body, 0)
        pltpu.sync_copy(rows_v, out_hbm.at[oi_v])

    return knn(pts_t)


def _cov_body(d_ref, s_ref, q_ref, o_ref):
    d1 = jnp.sqrt(d_ref[0:1, :])
    d2 = jnp.sqrt(d_ref[1:2, :])
    d3 = jnp.sqrt(d_ref[2:3, :])
    asc = jnp.maximum((d1 + d2 + d3) / 3.0, 1e-5)
    r = q_ref[0:1, :]
    x = q_ref[1:2, :]
    y = q_ref[2:3, :]
    z = q_ref[3:4, :]
    inv = 1.0 / jnp.sqrt(r * r + x * x + y * y + z * z)
    r = r * inv
    x = x * inv
    y = y * inv
    z = z * inv
    rot = (
        (1.0 - 2.0 * (y * y + z * z), 2.0 * (x * y - r * z), 2.0 * (x * z + r * y)),
        (2.0 * (x * y + r * z), 1.0 - 2.0 * (x * x + z * z), 2.0 * (y * z - r * x)),
        (2.0 * (x * z - r * y), 2.0 * (y * z + r * x), 1.0 - 2.0 * (x * x + y * y)),
    )
    p = tuple((s_ref[k : k + 1, :] * asc) ** 2 for k in range(3))
    for a in range(3):
        for b in range(a, 3):
            cov = p[0] * rot[a][0] * rot[b][0] + p[1] * rot[a][1] * rot[b][1] + p[2] * rot[a][2] * rot[b][2]
            o_ref[3 * a + b : 3 * a + b + 1, :] = cov
            if a != b:
                o_ref[3 * b + a : 3 * b + a + 1, :] = cov


def _cov_tc(d2top, scales_t, quats_t):
    return pl.pallas_call(
        _cov_body,
        out_shape=jax.ShapeDtypeStruct((9, N), jnp.float32),
    )(d2top, scales_t, quats_t)


def kernel(points, colors, scales, quaternions):
    del colors
    pts_t = points.T
    d2rows = _knn_sc(pts_t)
    d2top = d2rows[:, :3].T
    covt = _cov_tc(d2top, scales.T, quaternions.T)
    return covt.T.reshape(N, 3, 3)


# speedup vs baseline: 1.0885x; 1.0295x over previous
import dataclasses
import functools

import jax
import jax.numpy as jnp
from jax import lax
from jax.experimental import pallas as pl
from jax.experimental.pallas import tpu as pltpu
from jax.experimental.pallas import tpu_sc as plsc

N = 4096
NC = 2
NS = 16
L = 16
NW = NC * NS
QPW = N // NW
NB = 64
NCH = N // L
BPW = QPW // L
BIG = 1e18


def _knn_sc(pts_t):
    mesh = plsc.VectorSubcoreMesh(core_axis_name="c", subcore_axis_name="s")
    cp = pltpu.CompilerParams()
    if "needs_layout_passes" in pltpu.CompilerParams.__dataclass_fields__:
        cp = dataclasses.replace(cp, needs_layout_passes=False)

    @functools.partial(
        pl.kernel,
        mesh=mesh,
        compiler_params=cp,
        out_type=jax.ShapeDtypeStruct((N, 128), jnp.float32),
        scratch_types=[
            pltpu.VMEM((3, N), jnp.float32),
            pltpu.VMEM((N,), jnp.float32),
            pltpu.VMEM((N,), jnp.float32),
            pltpu.VMEM((N,), jnp.float32),
            pltpu.VMEM((N,), jnp.float32),
            pltpu.VMEM((N,), jnp.int32),
            pltpu.VMEM((N,), jnp.int32),
            pltpu.VMEM((NB * L,), jnp.int32),
            pltpu.VMEM((NB + L,), jnp.int32),
            pltpu.VMEM((QPW, 128), jnp.float32),
            pltpu.VMEM((QPW,), jnp.int32),
        ],
    )
    def knn(pts_hbm, out_hbm, pts_v, sx_v, sy_v, sz_v, spp_v, soi_v, bidx_v,
            cnt_v, s_v, rows_v, oi_v):
        wid = lax.axis_index("s") * NC + lax.axis_index("c")
        pltpu.sync_copy(pts_hbm, pts_v)
        lane = lax.iota(jnp.int32, L)
        inf = jnp.full((L,), jnp.inf, jnp.float32)
        ones_i = jnp.full((L,), 1, jnp.int32)

        def minmax_body(c, carry):
            mn, mx = carry
            v = pts_v[0, pl.ds(c * L, L)]
            return jnp.minimum(mn, v), jnp.maximum(mx, v)

        mn, mx = lax.fori_loop(0, NCH, minmax_body, (inf, -inf))
        xmin = jnp.full((L,), jnp.min(mn), jnp.float32)
        xmax = jnp.full((L,), jnp.max(mx), jnp.float32)
        denom = jnp.maximum(xmax - xmin, 1e-30)
        invw = jnp.float32(NB) / denom
        w = denom / jnp.float32(NB)
        margin = w * jnp.float32(1e-3)

        def zero_body(c, _):
            cnt_v[pl.ds(c * L, L)] = jnp.zeros((L,), jnp.int32)
            return 0

        lax.fori_loop(0, NB, zero_body, 0)

        def hist_body(c, _):
            xv = pts_v[0, pl.ds(c * L, L)]
            bi = ((xv - xmin) * invw).astype(jnp.int32)
            bi = jnp.minimum(jnp.maximum(bi, 0), NB - 1)
            bidx_v[pl.ds(c * L, L)] = bi
            plsc.addupdate_scatter(cnt_v, [bi * L + lane], ones_i)
            return 0

        lax.fori_loop(0, NCH, hist_body, 0)

        def pfx_body(c, run):
            v = cnt_v[pl.ds(c * L, L)]
            cs = plsc.cumsum(v)
            cnt_v[pl.ds(c * L, L)] = cs - v + run
            return run + cs[L - 1]

        lax.fori_loop(0, NB, pfx_body, jnp.int32(0))

        def sgather_body(g, _):
            idx = (g * L + lane) * L
            s_v[pl.ds(g * L, L)] = plsc.load_gather(cnt_v, [idx])
            return 0

        lax.fori_loop(0, NB // L, sgather_body, 0)
        s_v[pl.ds(NB, L)] = jnp.full((L,), N, jnp.int32)

        def place_body(c, _):
            bi = bidx_v[pl.ds(c * L, L)]
            cidx = bi * L + lane
            pos = plsc.load_gather(cnt_v, [cidx])
            px = pts_v[0, pl.ds(c * L, L)]
            py = pts_v[1, pl.ds(c * L, L)]
            pz = pts_v[2, pl.ds(c * L, L)]
            plsc.store_scatter(sx_v, [pos], px)
            plsc.store_scatter(sy_v, [pos], py)
            plsc.store_scatter(sz_v, [pos], pz)
            plsc.store_scatter(spp_v, [pos], px * px + py * py + pz * pz)
            plsc.store_scatter(soi_v, [pos], c * L + lane)
            plsc.store_scatter(cnt_v, [cidx], pos + 1)
            return 0

        lax.fori_loop(0, NCH, place_body, 0)

        def read_s(b):
            return plsc.load_gather(s_v, [jnp.full((L,), b, jnp.int32)])[0]

        def block_body(i, _):
            base = (i * NW + wid) * L
            qx = sx_v[pl.ds(base, L)]
            qy = sy_v[pl.ds(base, L)]
            qz = sz_v[pl.ds(base, L)]
            qlo = jnp.full((L,), jnp.min(qx), jnp.float32)
            qhi = jnp.full((L,), jnp.max(qx), jnp.float32)
            qx2 = qx * (-2.0)
            qy2 = qy * (-2.0)
            qz2 = qz * (-2.0)
            qq = qx * qx + qy * qy + qz * qz

            def insert3(t, d2):
                t1, t2, t3 = t
                b1 = jnp.maximum(t1, d2)
                t1 = jnp.minimum(t1, d2)
                b2 = jnp.maximum(t2, b1)
                t2 = jnp.minimum(t2, b1)
                t3 = jnp.minimum(t3, b2)
                return t1, t2, t3

            def scan(c0, c1, t):
                def chunk_body(c, t):
                    px = sx_v[pl.ds(c * L, L)]
                    py = sy_v[pl.ds(c * L, L)]
                    pz = sz_v[pl.ds(c * L, L)]
                    pp = spp_v[pl.ds(c * L, L)]
                    for k in range(L):
                        d2 = qx2 * px[k] + qy2 * py[k] + qz2 * pz[k] + pp[k]
                        t = insert3(t, d2)
                    return t

                return lax.fori_loop(c0, c1, chunk_body, t)

            def self_chunk(t):
                for k in range(L):
                    d2 = qx2 * qx[k] + qy2 * qy[k] + qz2 * qz[k] + qq[k]
                    d2 = jnp.where(lane == k, BIG, d2)
                    t = insert3(t, d2)
                return t

            def lane_bound(lo, hi):
                lov = jnp.broadcast_to(lo, (L,))
                hiv = jnp.broadcast_to(hi, (L,))
                edge_l = xmin + lov.astype(jnp.float32) * w
                edge_r = xmin + hiv.astype(jnp.float32) * w
                gap_l = jnp.where(lov == 0, BIG, qx - edge_l - margin)
                gap_r = jnp.where(hiv == NB, BIG, edge_r - qx - margin)
                g = jnp.maximum(jnp.minimum(gap_l, gap_r), 0.0)
                return g * g - qq

            lo0 = jnp.maximum(
                jnp.minimum(((qlo - xmin) * invw).astype(jnp.int32)[0],
                            NB - 1) - 1, 0)
            hi0 = jnp.minimum(
                jnp.minimum(((qhi - xmin) * invw).astype(jnp.int32)[0],
                            NB - 1) + 2, NB)
            lc0 = lax.shift_right_logical(read_s(lo0), 4)
            rc0 = jnp.minimum(
                lax.shift_right_logical(read_s(hi0) + (L - 1), 4), NCH)
            blk = i * NW + wid
            t = scan(lc0, blk, (inf, inf, inf))
            t = self_chunk(t)
            t = scan(blk + 1, rc0, t)

            def w_cond(carry):
                t1, t2, t3, lo, hi, lc, rc = carry
                return jnp.any(t3 > lane_bound(lo, hi))

            def w_body(carry):
                t1, t2, t3, lo, hi, lc, rc = carry
                nlo = jnp.maximum(lo - 1, 0)
                nhi = jnp.minimum(hi + 1, NB)
                nlc = lax.shift_right_logical(read_s(nlo), 4)
                nrc = jnp.minimum(
                    lax.shift_right_logical(read_s(nhi) + (L - 1), 4), NCH)
                t = scan(nlc, lc, (t1, t2, t3))
                t = scan(rc, nrc, t)
                return t + (nlo, nhi, nlc, nrc)

            t1, t2, t3, _, _, _, _ = lax.while_loop(
                w_cond, w_body, t + (lo0, hi0, lc0, rc0))

            row = i * L + lane
            plsc.store_scatter(rows_v, [row, jnp.zeros((L,), jnp.int32)],
                               t1 + qq)
            plsc.store_scatter(rows_v, [row, jnp.full((L,), 1, jnp.int32)],
                               t2 + qq)
            plsc.store_scatter(rows_v, [row, jnp.full((L,), 2, jnp.int32)],
                               t3 + qq)
            oi_v[pl.ds(i * L, L)] = soi_v[pl.ds(base, L)]
            return 0

        lax.fori_loop(0, BPW, block_body, 0)
        pltpu.sync_copy(rows_v, out_hbm.at[oi_v])

    return knn(pts_t)


def _cov_body(d_ref, s_ref, q_ref, o_ref):
    d1 = jnp.sqrt(d_ref[0:1, :])
    d2 = jnp.sqrt(d_ref[1:2, :])
    d3 = jnp.sqrt(d_ref[2:3, :])
    asc = jnp.maximum((d1 + d2 + d3) / 3.0, 1e-5)
    r = q_ref[0:1, :]
    x = q_ref[1:2, :]
    y = q_ref[2:3, :]
    z = q_ref[3:4, :]
    inv = 1.0 / jnp.sqrt(r * r + x * x + y * y + z * z)
    r = r * inv
    x = x * inv
    y = y * inv
    z = z * inv
    rot = (
        (1.0 - 2.0 * (y * y + z * z), 2.0 * (x * y - r * z), 2.0 * (x * z + r * y)),
        (2.0 * (x * y + r * z), 1.0 - 2.0 * (x * x + z * z), 2.0 * (y * z - r * x)),
        (2.0 * (x * z - r * y), 2.0 * (y * z + r * x), 1.0 - 2.0 * (x * x + y * y)),
    )
    p = tuple((s_ref[k : k + 1, :] * asc) ** 2 for k in range(3))
    for a in range(3):
        for b in range(a, 3):
            cov = p[0] * rot[a][0] * rot[b][0] + p[1] * rot[a][1] * rot[b][1] + p[2] * rot[a][2] * rot[b][2]
            o_ref[3 * a + b : 3 * a + b + 1, :] = cov
            if a != b:
                o_ref[3 * b + a : 3 * b + a + 1, :] = cov


def _cov_tc(d2top, scales_t, quats_t):
    return pl.pallas_call(
        _cov_body,
        out_shape=jax.ShapeDtypeStruct((9, N), jnp.float32),
    )(d2top, scales_t, quats_t)


def kernel(points, colors, scales, quaternions):
    del colors
    pts_t = points.T
    d2rows = _knn_sc(pts_t)
    d2top = d2rows[:, :3].T
    covt = _cov_tc(d2top, scales.T, quaternions.T)
    return covt.T.reshape(N, 3, 3)
